# baseline (speedup 1.0000x reference)
import functools

import jax
import jax.numpy as jnp
from jax import lax
from jax.experimental import pallas as pl
from jax.experimental.pallas import tpu as pltpu
from jax.experimental.pallas import tpu_sc as plsc

R = 128
N = 32768
K = 115
L = 16
NB = 16
NC = 2
NS = 16
NW = NC * NS
ROWS_PER_W = R // NW
G = 2048
NGV = G // L


def _loss_body(pred_ref, tgt_ref, w_ref, out_ref, gmax_ref):
    i = pl.program_id(0)
    p = pred_ref[...]
    t1 = tgt_ref[...] == 1
    w = w_ref[...]
    pt = jnp.where(t1, 1.0 - p, p)
    fw = jnp.where(t1, 0.25, 0.75) * (pt * pt)
    l1 = jnp.log1p(jnp.exp(-jnp.abs(p)))
    bce = jnp.maximum(jnp.where(t1, -p, p), 0.0) + l1
    loss = bce * w * fw
    out_ref[...] = loss

    @pl.when(i == 0)
    def _():
        gmax_ref[...] = loss

    @pl.when(i != 0)
    def _():
        gmax_ref[...] = jnp.maximum(gmax_ref[...], loss)


def _compute_loss(pred, target, weight):
    grid = (N // G,)
    return pl.pallas_call(
        _loss_body,
        out_shape=[
            jax.ShapeDtypeStruct((R, N), jnp.float32),
            jax.ShapeDtypeStruct((R, G), jnp.float32),
        ],
        grid=grid,
        in_specs=[
            pl.BlockSpec((R, G), lambda i: (0, i)),
            pl.BlockSpec((R, G), lambda i: (0, i)),
            pl.BlockSpec((R, G), lambda i: (0, i)),
        ],
        out_specs=[
            pl.BlockSpec((R, G), lambda i: (0, i)),
            pl.BlockSpec((R, G), lambda i: (0, 0)),
        ],
    )(pred, target, weight)


def _lane():
    return lax.iota(jnp.int32, L)


def _bytes_of(v, shift):
    u = plsc.bitcast(v, jnp.uint32)
    return ((u >> shift) & 0xFF).astype(jnp.int32)


def _zero_hist2(hcnt, hsum):
    @pl.loop(0, NB)
    def _(t):
        hcnt[pl.ds(t * L, L)] = jnp.zeros((L,), jnp.int32)
        hsum[pl.ds(t * L, L)] = jnp.zeros((L,), jnp.float32)


def _zero_hist1(hcnt):
    @pl.loop(0, NB)
    def _(t):
        hcnt[pl.ds(t * L, L)] = jnp.zeros((L,), jnp.int32)


def _hist_pass2(src, m, shift, hcnt, hsum):
    nv = m // L
    ones = jnp.ones((L,), jnp.int32)

    @pl.loop(0, nv)
    def _(j):
        v = src[pl.ds(j * L, L)]
        byte = _bytes_of(v, shift)
        plsc.addupdate_scatter(hcnt, [byte], ones)
        plsc.addupdate_scatter(hsum, [byte], v)

    if isinstance(m, int) and m % L == 0:
        return
    rem = m - nv * L
    msk = _lane() < rem
    v = src[pl.ds(nv * L, L)]
    byte = _bytes_of(v, shift)
    plsc.addupdate_scatter(hcnt, [byte], ones, mask=msk)
    plsc.addupdate_scatter(hsum, [byte], v, mask=msk)


def _hist_pass1(src, m, shift, hcnt):
    nv = m // L
    ones = jnp.ones((L,), jnp.int32)

    @pl.loop(0, nv)
    def _(j):
        v = src[pl.ds(j * L, L)]
        plsc.addupdate_scatter(hcnt, [_bytes_of(v, shift)], ones)

    if isinstance(m, int) and m % L == 0:
        return
    rem = m - nv * L
    msk = _lane() < rem
    v = src[pl.ds(nv * L, L)]
    plsc.addupdate_scatter(hcnt, [_bytes_of(v, shift)], ones, mask=msk)


def _select2(needed, hcnt, hsum):
    lane = _lane()

    def cond(carry):
        tt, _, _, f_b, _, _ = carry
        return (f_b < 0) & (tt < NB)

    def body(carry):
        tt, cum_c, cum_s, f_b, f_c, f_s = carry
        t = NB - 1 - tt
        cnt = hcnt[pl.ds(t * L, L)]
        sm = hsum[pl.ds(t * L, L)]
        sic = lax.rev(plsc.cumsum(lax.rev(cnt, (0,))), (0,))
        sis = lax.rev(plsc.cumsum(lax.rev(sm, (0,))), (0,))
        c_ab = cum_c + sic - cnt
        s_ab = cum_s + sis - sm
        hit = (c_ab < needed) & (c_ab + cnt >= needed)
        f_b = jnp.maximum(f_b, jnp.max(jnp.where(hit, t * L + lane, -1)))
        f_c = jnp.maximum(f_c, jnp.max(jnp.where(hit, c_ab, -1)))
        f_s = jnp.maximum(f_s, jnp.max(jnp.where(hit, s_ab, -1.0)))
        return tt + 1, cum_c + sic[0], cum_s + sis[0], f_b, f_c, f_s

    init = (jnp.int32(0), jnp.int32(0), jnp.float32(0.0), jnp.int32(-1),
            jnp.int32(-1), jnp.float32(-1.0))
    _, _, _, f_b, f_c, f_s = lax.while_loop(cond, body, init)
    return f_b, f_c, f_s


def _select1(needed, hcnt):
    lane = _lane()

    def cond(carry):
        tt, _, f_b, _ = carry
        return (f_b < 0) & (tt < NB)

    def body(carry):
        tt, cum_c, f_b, f_c = carry
        t = NB - 1 - tt
        cnt = hcnt[pl.ds(t * L, L)]
        sic = lax.rev(plsc.cumsum(lax.rev(cnt, (0,))), (0,))
        c_ab = cum_c + sic - cnt
        hit = (c_ab < needed) & (c_ab + cnt >= needed)
        f_b = jnp.maximum(f_b, jnp.max(jnp.where(hit, t * L + lane, -1)))
        f_c = jnp.maximum(f_c, jnp.max(jnp.where(hit, c_ab, -1)))
        return tt + 1, cum_c + sic[0], f_b, f_c

    init = (jnp.int32(0), jnp.int32(0), jnp.int32(-1), jnp.int32(-1))
    _, _, f_b, f_c = lax.while_loop(cond, body, init)
    return f_b, f_c


def _select_final(needed, prefix, hcnt, hsum):
    lane = _lane()

    def cond(carry):
        tt, _, _, f_t = carry
        return (f_t < 0.0) & (tt < NB)

    def body(carry):
        tt, cum_c, cum_s, f_t = carry
        t = NB - 1 - tt
        cnt = hcnt[pl.ds(t * L, L)]
        sm = hsum[pl.ds(t * L, L)]
        sic = lax.rev(plsc.cumsum(lax.rev(cnt, (0,))), (0,))
        sis = lax.rev(plsc.cumsum(lax.rev(sm, (0,))), (0,))
        c_ab = cum_c + sic - cnt
        s_ab = cum_s + sis - sm
        hit = (c_ab < needed) & (c_ab + cnt >= needed)
        val = plsc.bitcast(prefix | (t * L + lane), jnp.float32)
        term = s_ab + (needed - c_ab).astype(jnp.float32) * val
        f_t = jnp.maximum(f_t, jnp.max(jnp.where(hit, term, -1.0)))
        return tt + 1, cum_c + sic[0], cum_s + sis[0], f_t

    init = (jnp.int32(0), jnp.int32(0), jnp.float32(0.0), jnp.float32(-1.0))
    _, _, _, f_t = lax.while_loop(cond, body, init)
    return f_t


def _zero_h2(h2):
    @pl.loop(0, NB * L)
    def _(t):
        h2[pl.ds(t * L, L)] = jnp.zeros((L,), jnp.int32)


def _hist_pass_ls(src, m, shift, h2):
    lane = _lane()
    ones = jnp.ones((L,), jnp.int32)

    @pl.loop(0, m // L)
    def _(j):
        v = src[pl.ds(j * L, L)]
        byte = _bytes_of(v, shift)
        slot = byte * L + ((lane + byte) & (L - 1))
        plsc.addupdate_scatter(h2, [slot], ones)


def _select_ls(needed, h2):
    lane = _lane()

    def cond(carry):
        tt, _, f_b, _ = carry
        return (f_b < 0) & (tt < NB)

    def body(carry):
        tt, cum_c, f_b, f_c = carry
        t = NB - 1 - tt
        buckets = t * L + lane
        base = buckets * L
        cnt = jnp.zeros((L,), jnp.int32)
        for l in range(L):
            cnt = cnt + plsc.load_gather(
                h2, [base + ((l + buckets) & (L - 1))])
        sic = lax.rev(plsc.cumsum(lax.rev(cnt, (0,))), (0,))
        c_ab = cum_c + sic - cnt
        hit = (c_ab < needed) & (c_ab + cnt >= needed)
        f_b = jnp.maximum(f_b, jnp.max(jnp.where(hit, buckets, -1)))
        f_c = jnp.maximum(f_c, jnp.max(jnp.where(hit, c_ab, -1)))
        return tt + 1, cum_c + sic[0], f_b, f_c

    init = (jnp.int32(0), jnp.int32(0), jnp.int32(-1), jnp.int32(-1))
    _, _, f_b, f_c = lax.while_loop(cond, body, init)
    return f_b, f_c


def _compact(src, dst, m, shift, b):
    nv = m // L

    def body(j, off):
        v = src[pl.ds(j * L, L)]
        keep = _bytes_of(v, shift) == b
        plsc.store_compressed(dst.at[pl.ds(off, L)], v, mask=keep)
        return off + plsc.all_reduce_population_count(keep)[0]

    off = lax.fori_loop(0, nv, body, jnp.int32(0))
    if isinstance(m, int) and m % L == 0:
        return off
    rem = m - nv * L
    v = src[pl.ds(nv * L, L)]
    keep = (_bytes_of(v, shift) == b) & (_lane() < rem)
    plsc.store_compressed(dst.at[pl.ds(off, L)], v, mask=keep)
    return off + plsc.all_reduce_population_count(keep)[0]


def _phase_a_tau(gbuf, ga, hcnt, h2):
    _zero_h2(h2)
    _hist_pass_ls(gbuf, G, 24, h2)
    b3, c3 = _select_ls(K, h2)
    mg = _compact(gbuf, ga, G, 24, b3)
    _zero_hist1(hcnt)
    _hist_pass1(ga, mg, 16, hcnt)
    b2, _ = _select1(K - c3, hcnt)
    return (b3 << 24) | (b2 << 16)


def _phase_b_sum(rowbuf, abuf, gbuf, gid, hcnt, hsum, tau_bits):
    lane = _lane()
    tau_v = plsc.bitcast(jnp.zeros((L,), jnp.int32) + tau_bits, jnp.float32)

    def scan_body(j, off):
        v = gbuf[pl.ds(j * L, L)]
        keep = v >= tau_v
        plsc.store_compressed(gid.at[pl.ds(off, L)], j * L + lane, mask=keep)
        return off + plsc.all_reduce_population_count(keep)[0]

    ng = lax.fori_loop(0, NGV, scan_body, jnp.int32(0))

    gid[pl.ds(ng, L)] = jnp.zeros((L,), jnp.int32)
    _zero_hist2(hcnt, hsum)
    ones = jnp.ones((L,), jnp.int32)
    ngv_b = (ng + (L - 1)) // L

    @pl.loop(0, ngv_b)
    def _(gi):
        jv = gid[pl.ds(gi * L, L)]
        msk = (gi * L + lane) < ng
        for s in range(L):
            v = plsc.load_gather(rowbuf, [jv + (G * s)])
            v = jnp.where(msk, v, 0.0)
            abuf[pl.ds(gi * (L * L) + s * L, L)] = v
            byte = _bytes_of(v, 24)
            plsc.addupdate_scatter(hcnt, [byte], ones)
            plsc.addupdate_scatter(hsum, [byte], v)

    m = ngv_b * (L * L)
    needed = jnp.int32(K)
    b, c_ab, s_ab = _select2(needed, hcnt, hsum)
    acc = s_ab
    needed = needed - c_ab
    m = _compact(abuf, rowbuf, m, 24, b)
    prefix = b << 24
    for level in (1, 2):
        shift = 24 - 8 * level
        src = rowbuf if level == 1 else abuf
        dst = abuf if level == 1 else rowbuf
        _zero_hist2(hcnt, hsum)
        _hist_pass2(src, m, shift, hcnt, hsum)
        b, c_ab, s_ab = _select2(needed, hcnt, hsum)
        acc = acc + s_ab
        needed = needed - c_ab
        m = _compact(src, dst, m, shift, b)
        prefix = prefix | (b << shift)
    _zero_hist2(hcnt, hsum)
    _hist_pass2(rowbuf, m, 0, hcnt, hsum)
    return acc + _select_final(needed, prefix, hcnt, hsum)


def _sc_topk_sums(loss, gmax):
    mesh = plsc.VectorSubcoreMesh(core_axis_name="c", subcore_axis_name="s")

    @functools.partial(
        pl.kernel,
        out_type=jax.ShapeDtypeStruct((NW, L), jnp.float32),
        mesh=mesh,
        compiler_params=pltpu.CompilerParams(needs_layout_passes=False),
        scratch_types=[
            pltpu.VMEM((N + L,), jnp.float32),
            pltpu.VMEM((N + L,), jnp.float32),
            pltpu.VMEM((N + L,), jnp.float32),
            pltpu.VMEM((G,), jnp.float32),
            pltpu.VMEM((G,), jnp.float32),
            pltpu.VMEM((G + L,), jnp.float32),
            pltpu.VMEM((G + L,), jnp.int32),
            pltpu.VMEM((NB * L * L,), jnp.int32),
            pltpu.VMEM((256,), jnp.int32),
            pltpu.VMEM((256,), jnp.float32),
            pltpu.VMEM((L,), jnp.float32),
            pltpu.SemaphoreType.DMA,
            pltpu.SemaphoreType.DMA,
            pltpu.SemaphoreType.DMA,
            pltpu.SemaphoreType.DMA,
        ],
    )
    def k(loss_hbm, gmax_hbm, out_hbm, r0, r1, abuf, g0, g1, ga, gid, h2,
          hcnt, hsum, outv, semr0, semr1, semg0, semg1):
        wid = lax.axis_index("s") * NC + lax.axis_index("c")
        base = wid * ROWS_PER_W
        rbufs = [r0, r1]
        gbufs = [g0, g1]
        semr = [semr0, semr1]
        semg = [semg0, semg1]
        rdesc = [None] * ROWS_PER_W
        gdesc = [None] * ROWS_PER_W
        for i in range(2):
            rdesc[i] = pltpu.async_copy(
                loss_hbm.at[base + i], rbufs[i].at[pl.ds(0, N)], semr[i])
            gdesc[i] = pltpu.async_copy(
                gmax_hbm.at[base + i], gbufs[i], semg[i])
        sums = jnp.zeros((L,), jnp.float32)
        lane = _lane()
        for i in range(ROWS_PER_W):
            rdesc[i].wait()
            gdesc[i].wait()
            tau = _phase_a_tau(gbufs[i % 2], ga, hcnt, h2)
            acc = _phase_b_sum(rbufs[i % 2], abuf, gbufs[i % 2], gid,
                               hcnt, hsum, tau)
            if i + 2 < ROWS_PER_W:
                rdesc[i + 2] = pltpu.async_copy(
                    loss_hbm.at[base + i + 2], rbufs[i % 2].at[pl.ds(0, N)],
                    semr[i % 2])
                gdesc[i + 2] = pltpu.async_copy(
                    gmax_hbm.at[base + i + 2], gbufs[i % 2], semg[i % 2])
            sums = jnp.where(lane == i, acc, sums)
        outv[...] = sums
        pltpu.sync_copy(outv, out_hbm.at[wid])

    return k(loss, gmax)


def kernel(pred, target, weight):
    loss, gmax = _compute_loss(pred, target, weight)
    sums = _sc_topk_sums(loss, gmax)
    return jnp.sum(sums) / jnp.float32(R * K)


# speedup vs baseline: 1.0517x; 1.0517x over previous
import functools

import jax
import jax.numpy as jnp
from jax import lax
from jax.experimental import pallas as pl
from jax.experimental.pallas import tpu as pltpu
from jax.experimental.pallas import tpu_sc as plsc

R = 128
N = 32768
K = 115
L = 16
NB = 16
NC = 2
NS = 16
NW = NC * NS
ROWS_PER_W = R // NW
G = 2048
NGV = G // L


def _loss_body(pred_ref, tgt_ref, w_ref, out_ref, gmax_ref):
    i = pl.program_id(0)
    p = pred_ref[...]
    t1 = tgt_ref[...] == 1
    w = w_ref[...]
    pt = jnp.where(t1, 1.0 - p, p)
    fw = jnp.where(t1, 0.25, 0.75) * (pt * pt)
    l1 = jnp.log1p(jnp.exp(-jnp.abs(p)))
    bce = jnp.maximum(jnp.where(t1, -p, p), 0.0) + l1
    loss = bce * w * fw
    out_ref[...] = loss

    @pl.when(i == 0)
    def _():
        gmax_ref[...] = loss

    @pl.when(i != 0)
    def _():
        gmax_ref[...] = jnp.maximum(gmax_ref[...], loss)


def _compute_loss(pred, target, weight):
    grid = (N // G,)
    return pl.pallas_call(
        _loss_body,
        out_shape=[
            jax.ShapeDtypeStruct((R, N), jnp.float32),
            jax.ShapeDtypeStruct((R, G), jnp.float32),
        ],
        grid=grid,
        in_specs=[
            pl.BlockSpec((R, G), lambda i: (0, i)),
            pl.BlockSpec((R, G), lambda i: (0, i)),
            pl.BlockSpec((R, G), lambda i: (0, i)),
        ],
        out_specs=[
            pl.BlockSpec((R, G), lambda i: (0, i)),
            pl.BlockSpec((R, G), lambda i: (0, 0)),
        ],
    )(pred, target, weight)


def _lane():
    return lax.iota(jnp.int32, L)


def _bytes_of(v, shift):
    u = plsc.bitcast(v, jnp.uint32)
    return ((u >> shift) & 0xFF).astype(jnp.int32)


def _zero_hist2(hcnt, hsum):
    @pl.loop(0, NB)
    def _(t):
        hcnt[pl.ds(t * L, L)] = jnp.zeros((L,), jnp.int32)
        hsum[pl.ds(t * L, L)] = jnp.zeros((L,), jnp.float32)


def _zero_hist1(hcnt):
    @pl.loop(0, NB)
    def _(t):
        hcnt[pl.ds(t * L, L)] = jnp.zeros((L,), jnp.int32)


def _hist_pass2(src, m, shift, hcnt, hsum):
    nv = m // L
    ones = jnp.ones((L,), jnp.int32)

    @pl.loop(0, nv)
    def _(j):
        v = src[pl.ds(j * L, L)]
        byte = _bytes_of(v, shift)
        plsc.addupdate_scatter(hcnt, [byte], ones)
        plsc.addupdate_scatter(hsum, [byte], v)

    if isinstance(m, int) and m % L == 0:
        return
    rem = m - nv * L
    msk = _lane() < rem
    v = src[pl.ds(nv * L, L)]
    byte = _bytes_of(v, shift)
    plsc.addupdate_scatter(hcnt, [byte], ones, mask=msk)
    plsc.addupdate_scatter(hsum, [byte], v, mask=msk)


def _hist_pass1(src, m, shift, hcnt):
    nv = m // L
    ones = jnp.ones((L,), jnp.int32)

    @pl.loop(0, nv)
    def _(j):
        v = src[pl.ds(j * L, L)]
        plsc.addupdate_scatter(hcnt, [_bytes_of(v, shift)], ones)

    if isinstance(m, int) and m % L == 0:
        return
    rem = m - nv * L
    msk = _lane() < rem
    v = src[pl.ds(nv * L, L)]
    plsc.addupdate_scatter(hcnt, [_bytes_of(v, shift)], ones, mask=msk)


def _select2(needed, hcnt, hsum):
    lane = _lane()

    def cond(carry):
        tt, _, _, f_b, _, _ = carry
        return (f_b < 0) & (tt < NB)

    def body(carry):
        tt, cum_c, cum_s, f_b, f_c, f_s = carry
        t = NB - 1 - tt
        cnt = hcnt[pl.ds(t * L, L)]
        sm = hsum[pl.ds(t * L, L)]
        sic = lax.rev(plsc.cumsum(lax.rev(cnt, (0,))), (0,))
        sis = lax.rev(plsc.cumsum(lax.rev(sm, (0,))), (0,))
        c_ab = cum_c + sic - cnt
        s_ab = cum_s + sis - sm
        hit = (c_ab < needed) & (c_ab + cnt >= needed)
        f_b = jnp.maximum(f_b, jnp.max(jnp.where(hit, t * L + lane, -1)))
        f_c = jnp.maximum(f_c, jnp.max(jnp.where(hit, c_ab, -1)))
        f_s = jnp.maximum(f_s, jnp.max(jnp.where(hit, s_ab, -1.0)))
        return tt + 1, cum_c + sic[0], cum_s + sis[0], f_b, f_c, f_s

    init = (jnp.int32(0), jnp.int32(0), jnp.float32(0.0), jnp.int32(-1),
            jnp.int32(-1), jnp.float32(-1.0))
    _, _, _, f_b, f_c, f_s = lax.while_loop(cond, body, init)
    return f_b, f_c, f_s


def _select1(needed, hcnt):
    lane = _lane()

    def cond(carry):
        tt, _, f_b, _ = carry
        return (f_b < 0) & (tt < NB)

    def body(carry):
        tt, cum_c, f_b, f_c = carry
        t = NB - 1 - tt
        cnt = hcnt[pl.ds(t * L, L)]
        sic = lax.rev(plsc.cumsum(lax.rev(cnt, (0,))), (0,))
        c_ab = cum_c + sic - cnt
        hit = (c_ab < needed) & (c_ab + cnt >= needed)
        f_b = jnp.maximum(f_b, jnp.max(jnp.where(hit, t * L + lane, -1)))
        f_c = jnp.maximum(f_c, jnp.max(jnp.where(hit, c_ab, -1)))
        return tt + 1, cum_c + sic[0], f_b, f_c

    init = (jnp.int32(0), jnp.int32(0), jnp.int32(-1), jnp.int32(-1))
    _, _, f_b, f_c = lax.while_loop(cond, body, init)
    return f_b, f_c


def _select_final(needed, prefix, hcnt, hsum):
    lane = _lane()

    def cond(carry):
        tt, _, _, f_t = carry
        return (f_t < 0.0) & (tt < NB)

    def body(carry):
        tt, cum_c, cum_s, f_t = carry
        t = NB - 1 - tt
        cnt = hcnt[pl.ds(t * L, L)]
        sm = hsum[pl.ds(t * L, L)]
        sic = lax.rev(plsc.cumsum(lax.rev(cnt, (0,))), (0,))
        sis = lax.rev(plsc.cumsum(lax.rev(sm, (0,))), (0,))
        c_ab = cum_c + sic - cnt
        s_ab = cum_s + sis - sm
        hit = (c_ab < needed) & (c_ab + cnt >= needed)
        val = plsc.bitcast(prefix | (t * L + lane), jnp.float32)
        term = s_ab + (needed - c_ab).astype(jnp.float32) * val
        f_t = jnp.maximum(f_t, jnp.max(jnp.where(hit, term, -1.0)))
        return tt + 1, cum_c + sic[0], cum_s + sis[0], f_t

    init = (jnp.int32(0), jnp.int32(0), jnp.float32(0.0), jnp.float32(-1.0))
    _, _, _, f_t = lax.while_loop(cond, body, init)
    return f_t


def _zero_h2(h2):
    @pl.loop(0, NB * L)
    def _(t):
        h2[pl.ds(t * L, L)] = jnp.zeros((L,), jnp.int32)


def _hist_pass_ls(src, m, shift, h2):
    lane = _lane()
    ones = jnp.ones((L,), jnp.int32)

    @pl.loop(0, m // L)
    def _(j):
        v = src[pl.ds(j * L, L)]
        byte = _bytes_of(v, shift)
        slot = byte * L + ((lane + byte) & (L - 1))
        plsc.addupdate_scatter(h2, [slot], ones)


def _select_ls(needed, h2):
    lane = _lane()

    def cond(carry):
        tt, _, f_b, _ = carry
        return (f_b < 0) & (tt < NB)

    def body(carry):
        tt, cum_c, f_b, f_c = carry
        t = NB - 1 - tt
        buckets = t * L + lane
        base = buckets * L
        cnt = jnp.zeros((L,), jnp.int32)
        for l in range(L):
            cnt = cnt + plsc.load_gather(
                h2, [base + ((l + buckets) & (L - 1))])
        sic = lax.rev(plsc.cumsum(lax.rev(cnt, (0,))), (0,))
        c_ab = cum_c + sic - cnt
        hit = (c_ab < needed) & (c_ab + cnt >= needed)
        f_b = jnp.maximum(f_b, jnp.max(jnp.where(hit, buckets, -1)))
        f_c = jnp.maximum(f_c, jnp.max(jnp.where(hit, c_ab, -1)))
        return tt + 1, cum_c + sic[0], f_b, f_c

    init = (jnp.int32(0), jnp.int32(0), jnp.int32(-1), jnp.int32(-1))
    _, _, f_b, f_c = lax.while_loop(cond, body, init)
    return f_b, f_c


def _compact(src, dst, m, shift, b):
    nv = m // L

    def body(j, off):
        v = src[pl.ds(j * L, L)]
        keep = _bytes_of(v, shift) == b
        plsc.store_compressed(dst.at[pl.ds(off, L)], v, mask=keep)
        return off + plsc.all_reduce_population_count(keep)[0]

    off = lax.fori_loop(0, nv, body, jnp.int32(0))
    if isinstance(m, int) and m % L == 0:
        return off
    rem = m - nv * L
    v = src[pl.ds(nv * L, L)]
    keep = (_bytes_of(v, shift) == b) & (_lane() < rem)
    plsc.store_compressed(dst.at[pl.ds(off, L)], v, mask=keep)
    return off + plsc.all_reduce_population_count(keep)[0]


def _phase_a_tau(gbuf, ga, hcnt, h2):
    _zero_hist1(hcnt)
    _hist_pass1(gbuf, G, 24, hcnt)
    b3, c3 = _select1(K, hcnt)
    mg = _compact(gbuf, ga, G, 24, b3)
    _zero_hist1(hcnt)
    _hist_pass1(ga, mg, 16, hcnt)
    b2, _ = _select1(K - c3, hcnt)
    return (b3 << 24) | (b2 << 16)


def _phase_b_sum(rowbuf, abuf, gbuf, gid, hcnt, hsum, tau_bits):
    lane = _lane()
    tau_v = plsc.bitcast(jnp.zeros((L,), jnp.int32) + tau_bits, jnp.float32)

    def scan_body(j, off):
        v = gbuf[pl.ds(j * L, L)]
        keep = v >= tau_v
        plsc.store_compressed(gid.at[pl.ds(off, L)], j * L + lane, mask=keep)
        return off + plsc.all_reduce_population_count(keep)[0]

    ng = lax.fori_loop(0, NGV, scan_body, jnp.int32(0))

    gid[pl.ds(ng, L)] = jnp.zeros((L,), jnp.int32)
    _zero_hist2(hcnt, hsum)
    ones = jnp.ones((L,), jnp.int32)
    ngv_b = (ng + (L - 1)) // L

    @pl.loop(0, ngv_b)
    def _(gi):
        jv = gid[pl.ds(gi * L, L)]
        msk = (gi * L + lane) < ng
        for s in range(L):
            v = plsc.load_gather(rowbuf, [jv + (G * s)])
            v = jnp.where(msk, v, 0.0)
            abuf[pl.ds(gi * (L * L) + s * L, L)] = v
            byte = _bytes_of(v, 24)
            plsc.addupdate_scatter(hcnt, [byte], ones)
            plsc.addupdate_scatter(hsum, [byte], v)

    m = ngv_b * (L * L)
    needed = jnp.int32(K)
    b, c_ab, s_ab = _select2(needed, hcnt, hsum)
    acc = s_ab
    needed = needed - c_ab
    m = _compact(abuf, rowbuf, m, 24, b)
    prefix = b << 24
    for level in (1, 2):
        shift = 24 - 8 * level
        src = rowbuf if level == 1 else abuf
        dst = abuf if level == 1 else rowbuf
        _zero_hist2(hcnt, hsum)
        _hist_pass2(src, m, shift, hcnt, hsum)
        b, c_ab, s_ab = _select2(needed, hcnt, hsum)
        acc = acc + s_ab
        needed = needed - c_ab
        m = _compact(src, dst, m, shift, b)
        prefix = prefix | (b << shift)
    _zero_hist2(hcnt, hsum)
    _hist_pass2(rowbuf, m, 0, hcnt, hsum)
    return acc + _select_final(needed, prefix, hcnt, hsum)


def _sc_topk_sums(loss, gmax):
    mesh = plsc.VectorSubcoreMesh(core_axis_name="c", subcore_axis_name="s")

    @functools.partial(
        pl.kernel,
        out_type=jax.ShapeDtypeStruct((NW, L), jnp.float32),
        mesh=mesh,
        compiler_params=pltpu.CompilerParams(needs_layout_passes=False),
        scratch_types=[
            pltpu.VMEM((N + L,), jnp.float32),
            pltpu.VMEM((N + L,), jnp.float32),
            pltpu.VMEM((N + L,), jnp.float32),
            pltpu.VMEM((G,), jnp.float32),
            pltpu.VMEM((G,), jnp.float32),
            pltpu.VMEM((G + L,), jnp.float32),
            pltpu.VMEM((G + L,), jnp.int32),
            pltpu.VMEM((NB * L * L,), jnp.int32),
            pltpu.VMEM((256,), jnp.int32),
            pltpu.VMEM((256,), jnp.float32),
            pltpu.VMEM((L,), jnp.float32),
            pltpu.SemaphoreType.DMA,
            pltpu.SemaphoreType.DMA,
            pltpu.SemaphoreType.DMA,
            pltpu.SemaphoreType.DMA,
        ],
    )
    def k(loss_hbm, gmax_hbm, out_hbm, r0, r1, abuf, g0, g1, ga, gid, h2,
          hcnt, hsum, outv, semr0, semr1, semg0, semg1):
        wid = lax.axis_index("s") * NC + lax.axis_index("c")
        base = wid * ROWS_PER_W
        rbufs = [r0, r1]
        gbufs = [g0, g1]
        semr = [semr0, semr1]
        semg = [semg0, semg1]
        rdesc = [None] * ROWS_PER_W
        gdesc = [None] * ROWS_PER_W
        for i in range(2):
            rdesc[i] = pltpu.async_copy(
                loss_hbm.at[base + i], rbufs[i].at[pl.ds(0, N)], semr[i])
            gdesc[i] = pltpu.async_copy(
                gmax_hbm.at[base + i], gbufs[i], semg[i])
        sums = jnp.zeros((L,), jnp.float32)
        lane = _lane()
        for i in range(ROWS_PER_W):
            rdesc[i].wait()
            gdesc[i].wait()
            tau = _phase_a_tau(gbufs[i % 2], ga, hcnt, h2)
            acc = _phase_b_sum(rbufs[i % 2], abuf, gbufs[i % 2], gid,
                               hcnt, hsum, tau)
            if i + 2 < ROWS_PER_W:
                rdesc[i + 2] = pltpu.async_copy(
                    loss_hbm.at[base + i + 2], rbufs[i % 2].at[pl.ds(0, N)],
                    semr[i % 2])
                gdesc[i + 2] = pltpu.async_copy(
                    gmax_hbm.at[base + i + 2], gbufs[i % 2], semg[i % 2])
            sums = jnp.where(lane == i, acc, sums)
        outv[...] = sums
        pltpu.sync_copy(outv, out_hbm.at[wid])

    return k(loss, gmax)


def kernel(pred, target, weight):
    loss, gmax = _compute_loss(pred, target, weight)
    sums = _sc_topk_sums(loss, gmax)
    return jnp.sum(sums) / jnp.float32(R * K)
